# Initial kernel scaffold; baseline (speedup 1.0000x reference)
import jax
import jax.numpy as jnp
from jax.experimental import pallas as pl


def kernel(node_feat_raw, params, node2graph):
    raise NotImplementedError("write your pallas kernel here")


# speedup vs baseline: 5.2666x; 5.2666x over previous
import jax
import jax.numpy as jnp
from jax.experimental import pallas as pl

_B = 16
_N = 32768
_DIN = 64
_ND = 64
_GD = 64
_HID = 128
_NTY = 9
_NA = _B * _NTY
_TN = 2048
_NT = _N // _TN
_R = _N // 128

_F32 = jnp.float32
_I32 = jnp.int32


def _dot(a, b):
    return jnp.dot(a, b, preferred_element_type=_F32)


def _dot_hi(a, b):
    return jnp.dot(a, b, preferred_element_type=_F32,
                   precision=jax.lax.Precision.HIGHEST)


def _stage_a(x_ref, n2gr_ref, wfe_ref, bfe_ref, wge_ref, bge_ref,
             nf_ref, segsum_ref, counts_ref):
    i = pl.program_id(0)
    x = x_ref[...]
    h = jnp.maximum(_dot(x, wfe_ref[...]) + bfe_ref[...], 0.0)
    nf = jnp.maximum(_dot(h, wge_ref[...]) + bge_ref[...], 0.0)
    nf_ref[...] = nf
    n2g_row = n2gr_ref[0]
    gidx = jax.lax.broadcasted_iota(_I32, (_B, _TN), 0)
    maskT = (gidx == n2g_row).astype(_F32)
    part = _dot_hi(maskT, nf)
    cnt = jnp.sum(maskT, axis=1, keepdims=True)

    @pl.when(i == 0)
    def _():
        segsum_ref[...] = jnp.zeros_like(segsum_ref)
        counts_ref[...] = jnp.zeros_like(counts_ref)

    segsum_ref[...] += part
    counts_ref[...] += cnt


def _stage_b(nf_ref, n2gc_ref, segsum_ref, wg_ref, bg_ref,
             w1_ref, b1_ref, w2_ref, b2_ref,
             s1_ref, ms1_ref, gf_ref):
    i = pl.program_id(0)
    gf = _dot(segsum_ref[...], wg_ref[...]) + bg_ref[...]
    gf_ref[...] = gf
    nf = nf_ref[...]
    lane_g = jax.lax.broadcasted_iota(_I32, (_TN, _B), 1)
    mask_b = n2gc_ref[...] == lane_g
    gfn = _dot(mask_b.astype(_F32), gf)
    x1 = jnp.concatenate([nf, gfn], axis=1)
    hh = jnp.maximum(_dot(x1, w1_ref[...]) + b1_ref[...], 0.0)
    s1 = _dot(hh, w2_ref[...]) + b2_ref[...]
    s1_ref[...] = s1

    @pl.when(i == 0)
    def _():
        ms1_ref[...] = jnp.full_like(ms1_ref, -jnp.inf)

    seg = jnp.where(mask_b, s1, -jnp.inf)
    ms1_ref[...] = jnp.maximum(ms1_ref[...], jnp.max(seg, axis=0, keepdims=True))


def _stage_c(s1_ref, n2g2_ref, ms1_ref, m1_ref, idx1_ref):
    s = s1_ref[...]
    seg = n2g2_ref[...]
    rows = jax.lax.broadcasted_iota(_I32, s.shape, 0)
    cols = jax.lax.broadcasted_iota(_I32, s.shape, 1)
    flat = rows * 128 + cols

    m_node = jnp.zeros_like(s)
    mlist = []
    for b in range(_B):
        mb = ms1_ref[0, b]
        mb = jnp.where(jnp.isfinite(mb), mb, 0.0)
        mlist.append(mb)
        m_node = m_node + jnp.where(seg == b, mb, 0.0)
    d = s - m_node
    e = jnp.exp(d)
    lzlist = [jnp.log(jnp.sum(jnp.where(seg == b, e, 0.0))) for b in range(_B)]
    lz_node = jnp.zeros_like(s)
    for b in range(_B):
        lz_node = lz_node + jnp.where(seg == b, lzlist[b], 0.0)
    lp = d - lz_node

    riota = jax.lax.broadcasted_iota(_I32, (_B, 1), 0)
    m1v = jnp.zeros((_B, 1), _F32)
    idx1v = jnp.zeros((_B, 1), _I32)
    for b in range(_B):
        mask = seg == b
        m1b = jnp.max(jnp.where(mask, lp, -jnp.inf))
        eq = jnp.logical_and(mask, lp == m1b)
        ib = jnp.min(jnp.where(eq, flat, _N))
        m1v = m1v + jnp.where(riota == b, m1b, 0.0)
        idx1v = idx1v + jnp.where(riota == b, ib, 0)
    m1_ref[...] = m1v
    idx1_ref[...] = idx1v


def _stage_d(nf_ref, n2gc_ref, gf_ref, n1f_ref,
             w1_ref, b1_ref, w2_ref, b2_ref,
             s2_ref, ms2_ref):
    i = pl.program_id(0)
    nf = nf_ref[...]
    lane_g = jax.lax.broadcasted_iota(_I32, (_TN, _B), 1)
    mask_b = n2gc_ref[...] == lane_g
    maskf = mask_b.astype(_F32)
    gfn = _dot(maskf, gf_ref[...])
    n1fn = _dot(maskf, n1f_ref[...])
    x2 = jnp.concatenate([nf, gfn, n1fn], axis=1)
    hh = jnp.maximum(_dot(x2, w1_ref[...]) + b1_ref[...], 0.0)
    s2 = _dot(hh, w2_ref[...]) + b2_ref[...]
    s2_ref[...] = s2

    @pl.when(i == 0)
    def _():
        ms2_ref[...] = jnp.full_like(ms2_ref, -jnp.inf)

    seg = jnp.where(mask_b, s2, -jnp.inf)
    ms2_ref[...] = jnp.maximum(ms2_ref[...], jnp.max(seg, axis=0, keepdims=True))


def _stage_e1(s2_ref, n2g2_ref, ms2_ref, aemb_ref, gf_ref, n1f_ref,
              w1_ref, b1_ref, w2_ref, b2_ref,
              m2_ref, idx2_ref):
    r0_16 = jax.lax.broadcasted_iota(_I32, (_NA, _B), 0)
    r1_16 = jax.lax.broadcasted_iota(_I32, (_NA, _B), 1)
    rsel = jnp.logical_and(_NTY * r1_16 <= r0_16,
                           r0_16 < _NTY * r1_16 + _NTY).astype(_F32)
    g_row = jnp.zeros((_NA, 1), _I32)
    rr = jax.lax.broadcasted_iota(_I32, (_NA, 1), 0)
    for b in range(_B):
        g_row = g_row + jnp.where(
            jnp.logical_and(_NTY * b <= rr, rr < _NTY * b + _NTY), b, 0)
    t0_9 = jax.lax.broadcasted_iota(_I32, (_NA, _NTY), 0)
    t1_9 = jax.lax.broadcasted_iota(_I32, (_NA, _NTY), 1)
    tsel = ((t0_9 - _NTY * g_row) == t1_9).astype(_F32)
    a144 = _dot(tsel, aemb_ref[...])
    g144 = _dot(rsel, gf_ref[...])
    f144 = _dot(rsel, n1f_ref[...])
    xa = jnp.concatenate([a144, g144, f144], axis=1)
    ha = jnp.maximum(_dot(xa, w1_ref[...]) + b1_ref[...], 0.0)
    s2a = _dot(ha, w2_ref[...]) + b2_ref[...]

    s = s2_ref[...]
    seg = n2g2_ref[...]
    rows = jax.lax.broadcasted_iota(_I32, s.shape, 0)
    cols = jax.lax.broadcasted_iota(_I32, s.shape, 1)
    flat = rows * 128 + cols

    mlist = []
    m_node = jnp.zeros_like(s)
    for b in range(_B):
        sa_b = s2a[b * _NTY:(b + 1) * _NTY, :]
        mb = jnp.maximum(ms2_ref[0, b], jnp.max(sa_b))
        mb = jnp.where(jnp.isfinite(mb), mb, 0.0)
        mlist.append(mb)
        m_node = m_node + jnp.where(seg == b, mb, 0.0)
    d = s - m_node
    e = jnp.exp(d)
    lzlist = []
    for b in range(_B):
        sa_b = s2a[b * _NTY:(b + 1) * _NTY, :]
        zb = (jnp.sum(jnp.where(seg == b, e, 0.0))
              + jnp.sum(jnp.exp(sa_b - mlist[b])))
        lzlist.append(jnp.log(zb))
    lz_node = jnp.zeros_like(s)
    for b in range(_B):
        lz_node = lz_node + jnp.where(seg == b, lzlist[b], 0.0)
    lp = d - lz_node

    tio = jax.lax.broadcasted_iota(_I32, (_NTY, 1), 0)
    riota = jax.lax.broadcasted_iota(_I32, (_B, 1), 0)
    m2v = jnp.zeros((_B, 1), _F32)
    idx2v = jnp.zeros((_B, 1), _I32)
    big = _N + _NA
    for b in range(_B):
        sa_b = s2a[b * _NTY:(b + 1) * _NTY, :]
        lpa = (sa_b - mlist[b]) - lzlist[b]
        mask = seg == b
        m2b = jnp.maximum(jnp.max(jnp.where(mask, lp, -jnp.inf)), jnp.max(lpa))
        eq = jnp.logical_and(mask, lp == m2b)
        ir = jnp.min(jnp.where(eq, flat, big))
        ia = jnp.min(jnp.where(lpa == m2b, _N + b * _NTY + tio, big))
        ib = jnp.minimum(ir, ia)
        m2v = m2v + jnp.where(riota == b, m2b, 0.0)
        idx2v = idx2v + jnp.where(riota == b, ib, 0)
    m2_ref[...] = m2v
    idx2_ref[...] = idx2v


def _stage_e2(gf_ref, n1f_ref, ext2_ref, m1_ref, m2_ref, idx1_ref, idx2_ref,
              counts_ref, ew1, eb1, ew2, eb2, sw1, sb1, sw2, sb2,
              cw1, cb1, cw2, cb2,
              act_ref, val_ref, lp_ref):
    gf = gf_ref[...]

    def _head(x, w1, b1, w2, b2):
        h = jnp.maximum(_dot(x, w1[...]) + b1[...], 0.0)
        return _dot(h, w2[...]) + b2[...]

    def _lsm_pick(logits, width):
        mx = jnp.max(logits, axis=1, keepdims=True)
        sh = logits - mx
        lsm = sh - jnp.log(jnp.sum(jnp.exp(sh), axis=1, keepdims=True))
        lane = jax.lax.broadcasted_iota(_I32, lsm.shape, 1)
        amx = jnp.max(lsm, axis=1, keepdims=True)
        act = jnp.min(jnp.where(lsm == amx, lane, width), axis=1, keepdims=True)
        picked = jnp.sum(jnp.where(lane == act, lsm, 0.0), axis=1, keepdims=True)
        return act, picked

    xe = jnp.concatenate([n1f_ref[...], ext2_ref[...]], axis=1)
    a3, elp = _lsm_pick(_head(xe, ew1, eb1, ew2, eb2), 3)
    a4, slp = _lsm_pick(_head(gf, sw1, sb1, sw2, sb2), 2)
    vals = _head(gf, cw1, cb1, cw2, cb2)

    riota0 = jax.lax.broadcasted_iota(_I32, (_B, 1), 0)
    offs = jnp.zeros((_B, 1), _I32)
    total = jnp.zeros((), _I32)
    for b in range(_B):
        offs = offs + jnp.where(riota0 == b, total, 0)
        total = total + counts_ref[b, 0].astype(_I32)
    n_i = counts_ref[...].astype(_I32)
    idx1v = idx1_ref[...]
    idx2v = idx2_ref[...]
    riota = jax.lax.broadcasted_iota(_I32, (_B, 1), 0)
    a1 = idx1v - offs
    ta = idx2v - _N - _NTY * riota
    a2 = jnp.where(idx2v < _N, idx2v - offs, ta + n_i)
    act_ref[...] = jnp.concatenate([a1, a2, a3, a4], axis=1)
    val_ref[...] = vals
    lp_ref[...] = m1_ref[...] + m2_ref[...] + elp + slp


def _full(spec):
    return pl.BlockSpec(spec, lambda i: tuple(0 for _ in spec))


def _full0(spec):
    return pl.BlockSpec(spec, lambda: tuple(0 for _ in spec))


def kernel(node_feat_raw, params, node2graph):
    p = params
    x = node_feat_raw
    n2g = node2graph.astype(_I32)
    n2g_row3 = n2g.reshape(_NT, 1, _TN)
    n2g_col = n2g.reshape(_N, 1)
    n2g_2d = n2g.reshape(_R, 128)
    row = lambda v: v.reshape(1, -1)

    nf, segsum, counts = pl.pallas_call(
        _stage_a,
        grid=(_NT,),
        in_specs=[
            pl.BlockSpec((_TN, _DIN), lambda i: (i, 0)),
            pl.BlockSpec((1, 1, _TN), lambda i: (i, 0, 0)),
            _full((_DIN, _ND)), _full((1, _ND)),
            _full((_ND, _ND)), _full((1, _ND)),
        ],
        out_specs=[
            pl.BlockSpec((_TN, _ND), lambda i: (i, 0)),
            _full((_B, _ND)),
            _full((_B, 1)),
        ],
        out_shape=[
            jax.ShapeDtypeStruct((_N, _ND), _F32),
            jax.ShapeDtypeStruct((_B, _ND), _F32),
            jax.ShapeDtypeStruct((_B, 1), _F32),
        ],
    )(x, n2g_row3, p['W_fe'], row(p['b_fe']), p['W_ge'], row(p['b_ge']))

    s1, ms1, gf = pl.pallas_call(
        _stage_b,
        grid=(_NT,),
        in_specs=[
            pl.BlockSpec((_TN, _ND), lambda i: (i, 0)),
            pl.BlockSpec((_TN, 1), lambda i: (i, 0)),
            _full((_B, _ND)),
            _full((_ND, _GD)), _full((1, _GD)),
            _full((_ND + _GD, _HID)), _full((1, _HID)),
            _full((_HID, 1)), _full((1, 1)),
        ],
        out_specs=[
            pl.BlockSpec((_TN, 1), lambda i: (i, 0)),
            _full((1, _B)),
            _full((_B, _GD)),
        ],
        out_shape=[
            jax.ShapeDtypeStruct((_N, 1), _F32),
            jax.ShapeDtypeStruct((1, _B), _F32),
            jax.ShapeDtypeStruct((_B, _GD), _F32),
        ],
    )(nf, n2g_col, segsum, p['W_g'], row(p['b_g']),
      p['n1_W1'], row(p['n1_b1']), p['n1_W2'], row(p['n1_b2']))

    m1, idx1 = pl.pallas_call(
        _stage_c,
        in_specs=[_full0((_R, 128)), _full0((_R, 128)), _full0((1, _B))],
        out_specs=[_full0((_B, 1)), _full0((_B, 1))],
        out_shape=[jax.ShapeDtypeStruct((_B, 1), _F32),
                   jax.ShapeDtypeStruct((_B, 1), _I32)],
    )(s1.reshape(_R, 128), n2g_2d, ms1)

    idx1f = jnp.clip(idx1.reshape(-1), 0, _N - 1)
    n1f = jnp.take(nf, idx1f, axis=0)

    s2, ms2 = pl.pallas_call(
        _stage_d,
        grid=(_NT,),
        in_specs=[
            pl.BlockSpec((_TN, _ND), lambda i: (i, 0)),
            pl.BlockSpec((_TN, 1), lambda i: (i, 0)),
            _full((_B, _GD)), _full((_B, _ND)),
            _full((2 * _ND + _GD, _HID)), _full((1, _HID)),
            _full((_HID, 1)), _full((1, 1)),
        ],
        out_specs=[
            pl.BlockSpec((_TN, 1), lambda i: (i, 0)),
            _full((1, _B)),
        ],
        out_shape=[
            jax.ShapeDtypeStruct((_N, 1), _F32),
            jax.ShapeDtypeStruct((1, _B), _F32),
        ],
    )(nf, n2g_col, gf, n1f,
      p['n2_W1'], row(p['n2_b1']), p['n2_W2'], row(p['n2_b2']))

    m2, idx2 = pl.pallas_call(
        _stage_e1,
        in_specs=[
            _full0((_R, 128)), _full0((_R, 128)), _full0((1, _B)),
            _full0((_NTY, _ND)), _full0((_B, _GD)), _full0((_B, _ND)),
            _full0((2 * _ND + _GD, _HID)), _full0((1, _HID)),
            _full0((_HID, 1)), _full0((1, 1)),
        ],
        out_specs=[_full0((_B, 1)), _full0((_B, 1))],
        out_shape=[jax.ShapeDtypeStruct((_B, 1), _F32),
                   jax.ShapeDtypeStruct((_B, 1), _I32)],
    )(s2.reshape(_R, 128), n2g_2d, ms2, p['atom_emb'], gf, n1f,
      p['n2_W1'], row(p['n2_b1']), p['n2_W2'], row(p['n2_b2']))

    i2 = idx2.reshape(-1)
    is_node = i2 < _N
    nrow = jnp.take(nf, jnp.clip(i2, 0, _N - 1), axis=0)
    t2 = jnp.clip(i2 - _N - _NTY * jnp.arange(_B, dtype=_I32), 0, _NTY - 1)
    arow = jnp.take(p['atom_emb'], t2, axis=0)
    ext2 = jnp.where(is_node[:, None], nrow, arow)

    actions, values, log_probs = pl.pallas_call(
        _stage_e2,
        in_specs=[
            _full0((_B, _GD)), _full0((_B, _ND)), _full0((_B, _ND)),
            _full0((_B, 1)), _full0((_B, 1)), _full0((_B, 1)), _full0((_B, 1)),
            _full0((_B, 1)),
            _full0((2 * _ND, _HID)), _full0((1, _HID)), _full0((_HID, 3)), _full0((1, 3)),
            _full0((_GD, _HID)), _full0((1, _HID)), _full0((_HID, 2)), _full0((1, 2)),
            _full0((_GD, _HID)), _full0((1, _HID)), _full0((_HID, 1)), _full0((1, 1)),
        ],
        out_specs=[_full0((_B, 4)), _full0((_B, 1)), _full0((_B, 1))],
        out_shape=[jax.ShapeDtypeStruct((_B, 4), _I32),
                   jax.ShapeDtypeStruct((_B, 1), _F32),
                   jax.ShapeDtypeStruct((_B, 1), _F32)],
    )(gf, n1f, ext2, m1, m2, idx1, idx2, counts,
      p['e_W1'], row(p['e_b1']), p['e_W2'], row(p['e_b2']),
      p['s_W1'], row(p['s_b1']), p['s_W2'], row(p['s_b2']),
      p['c_W1'], row(p['c_b1']), p['c_W2'], row(p['c_b2']))

    return actions, values.reshape(-1), log_probs.reshape(-1)
